# Initial kernel scaffold; baseline (speedup 1.0000x reference)
import jax
import jax.numpy as jnp
from jax.experimental import pallas as pl


def kernel(x, edge_index, edge_attr, params):
    raise NotImplementedError("write your pallas kernel here")


# speedup vs baseline: 3.9387x; 3.9387x over previous
import functools

import jax
import jax.numpy as jnp
from jax import lax
from jax.experimental import pallas as pl
from jax.experimental.pallas import tpu as pltpu
from jax.experimental.pallas import tpu_sc as plsc

NC = 2
NS = 16
NW = NC * NS
CHUNK = 128

F32 = jnp.float32


def _sc_mesh():
    return plsc.VectorSubcoreMesh(core_axis_name="c", subcore_axis_name="s",
                                  num_cores=NC, num_subcores=NS)


def _sc_segsum(feat, src_p, dst_p, n_rows):
    ep = src_p.shape[0]
    n_per_w = ep // NW
    chunks = n_per_w // CHUNK
    assert n_per_w % CHUNK == 0 and n_rows % (NS * 8) == 0
    rpt = n_rows // NS
    zeros128 = jnp.zeros((n_rows, 128), F32)

    def body(feat_h, src_h, dst_h, z128_h, outa_h, sidx, didx, rows, acc_a,
             sem):
        cid = lax.axis_index("c")
        sid = lax.axis_index("s")
        wid = sid * NC + cid
        r0 = sid * rpt

        pltpu.sync_copy(z128_h.at[pl.ds(r0, rpt)], acc_a.at[pl.ds(r0, rpt)])
        plsc.subcore_barrier()

        base_w = wid * n_per_w

        def step(j, carry):
            base = base_w + j * CHUNK
            pltpu.sync_copy(src_h.at[pl.ds(base, CHUNK)], sidx)
            pltpu.sync_copy(dst_h.at[pl.ds(base, CHUNK)], didx)
            pltpu.async_copy(feat_h.at[sidx], rows, sem).wait()
            pltpu.sync_copy(rows, acc_a.at[didx], add=True)
            return carry

        lax.fori_loop(0, chunks, step, 0)
        plsc.subcore_barrier()

        pltpu.sync_copy(acc_a.at[pl.ds(r0, rpt)],
                        outa_h.at[cid, pl.ds(r0, rpt)])

    kern = pl.kernel(
        body,
        out_type=[jax.ShapeDtypeStruct((NC, n_rows, 128), F32)],
        mesh=_sc_mesh(),
        scratch_types=[
            pltpu.VMEM((CHUNK,), jnp.int32),
            pltpu.VMEM((CHUNK,), jnp.int32),
            pltpu.VMEM((CHUNK, 128), F32),
            pltpu.VMEM_SHARED((n_rows, 128), F32),
            pltpu.SemaphoreType.DMA,
        ],
    )
    return kern(feat, src_p, dst_p, zeros128)[0]


def _sc_edge_segsum(ea128, dst_p, n_rows):
    ep = dst_p.shape[0]
    n_per_w = ep // NW
    chunks = n_per_w // CHUNK
    rpt = n_rows // NS
    zeros128 = jnp.zeros((n_rows, 128), F32)

    def body(ea_h, dst_h, z128_h, oute_h, didx, eat, acc_e, sem):
        cid = lax.axis_index("c")
        sid = lax.axis_index("s")
        wid = sid * NC + cid
        r0 = sid * rpt
        pltpu.sync_copy(z128_h.at[pl.ds(r0, rpt)], acc_e.at[pl.ds(r0, rpt)])
        plsc.subcore_barrier()
        base_w = wid * n_per_w

        def step(j, carry):
            base = base_w + j * CHUNK
            pltpu.sync_copy(dst_h.at[pl.ds(base, CHUNK)], didx)
            pltpu.sync_copy(ea_h.at[pl.ds(base, CHUNK)], eat)
            pltpu.sync_copy(eat, acc_e.at[didx], add=True)
            return carry

        lax.fori_loop(0, chunks, step, 0)
        plsc.subcore_barrier()
        pltpu.sync_copy(acc_e.at[pl.ds(r0, rpt)],
                        oute_h.at[cid, pl.ds(r0, rpt)])

    kern = pl.kernel(
        body,
        out_type=[jax.ShapeDtypeStruct((NC, n_rows, 128), F32)],
        mesh=_sc_mesh(),
        scratch_types=[
            pltpu.VMEM((CHUNK,), jnp.int32),
            pltpu.VMEM((CHUNK, 128), F32),
            pltpu.VMEM_SHARED((n_rows, 128), F32),
            pltpu.SemaphoreType.DMA,
        ],
    )
    return kern(ea128, dst_p, zeros128)[0]


TC_BLK = 1000


def _tc_layer1(sa, ea, x, wl, we, wr, b, n):
    def body(sa_ref, ea_ref, x_ref, wl_ref, we_ref, wr_ref, b_ref, out_ref):
        s = sa_ref[0] + sa_ref[1]
        e = ea_ref[0] + ea_ref[1]
        inv = 1.0 / jnp.maximum(e[:, 16:17], 1.0)
        h = (
            jnp.dot(s * inv, wl_ref[...], preferred_element_type=F32,
                    precision=lax.Precision.HIGHEST)
            + jnp.dot(e[:, :16] * inv, we_ref[...], preferred_element_type=F32,
                      precision=lax.Precision.HIGHEST)
            + jnp.dot(x_ref[...], wr_ref[...], preferred_element_type=F32,
                      precision=lax.Precision.HIGHEST)
            + b_ref[...]
        )
        out_ref[...] = jnp.maximum(h, 0.0)

    full = lambda *shape: pl.BlockSpec(shape, lambda i: (0,) * len(shape))
    return pl.pallas_call(
        body,
        grid=(n // TC_BLK,),
        in_specs=[
            pl.BlockSpec((2, TC_BLK, 128), lambda i: (0, i, 0)),
            pl.BlockSpec((2, TC_BLK, 128), lambda i: (0, i, 0)),
            pl.BlockSpec((TC_BLK, 128), lambda i: (i, 0)),
            full(128, 128), full(16, 128), full(128, 128), full(128),
        ],
        out_specs=pl.BlockSpec((TC_BLK, 128), lambda i: (i, 0)),
        out_shape=jax.ShapeDtypeStruct((n, 128), F32),
    )(sa, ea, x, wl, we, wr, b)


def _tc_layer2_attn(sb, ea, h1, wl, we, wr, b, w1, b1, w2, n):
    def body(sb_ref, ea_ref, h1_ref, wl_ref, we_ref, wr_ref, b_ref,
             w1_ref, b1_ref, w2_ref, out_ref):
        s = sb_ref[0] + sb_ref[1]
        e = ea_ref[0] + ea_ref[1]
        inv = 1.0 / jnp.maximum(e[:, 16:17], 1.0)
        h2 = (
            jnp.dot(s * inv, wl_ref[...], preferred_element_type=F32,
                    precision=lax.Precision.HIGHEST)
            + jnp.dot(e[:, :16] * inv, we_ref[...], preferred_element_type=F32,
                      precision=lax.Precision.HIGHEST)
            + jnp.dot(h1_ref[...], wr_ref[...], preferred_element_type=F32,
                      precision=lax.Precision.HIGHEST)
            + b_ref[...]
        )
        h2a = h2[:, :64]
        h2b = h2[:, 64:]
        ua = jnp.tanh(jnp.dot(h2a, w1_ref[...], preferred_element_type=F32,
                              precision=lax.Precision.HIGHEST) + b1_ref[...])
        ub = jnp.tanh(jnp.dot(h2b, w1_ref[...], preferred_element_type=F32,
                              precision=lax.Precision.HIGHEST) + b1_ref[...])
        sa_s = jnp.dot(ua, w2_ref[...], preferred_element_type=F32,
                       precision=lax.Precision.HIGHEST)
        sb_s = jnp.dot(ub, w2_ref[...], preferred_element_type=F32,
                       precision=lax.Precision.HIGHEST)
        m = jnp.maximum(sa_s, sb_s)
        ea_w = jnp.exp(sa_s - m)
        eb_w = jnp.exp(sb_s - m)
        out_ref[...] = (ea_w * h2a + eb_w * h2b) / (ea_w + eb_w)

    full = lambda *shape: pl.BlockSpec(shape, lambda i: (0,) * len(shape))
    return pl.pallas_call(
        body,
        grid=(n // TC_BLK,),
        in_specs=[
            pl.BlockSpec((2, TC_BLK, 128), lambda i: (0, i, 0)),
            pl.BlockSpec((2, TC_BLK, 128), lambda i: (0, i, 0)),
            pl.BlockSpec((TC_BLK, 128), lambda i: (i, 0)),
            full(128, 128), full(16, 128), full(128, 128), full(128),
            full(64, 64), full(64), full(64, 1),
        ],
        out_specs=pl.BlockSpec((TC_BLK, 64), lambda i: (i, 0)),
        out_shape=jax.ShapeDtypeStruct((n, 64), F32),
    )(sb, ea, h1, wl, we, wr, b, w1, b1, w2)


def _block_diag(a, b):
    da0, da1 = a.shape
    db0, db1 = b.shape
    out = jnp.zeros((da0 + db0, da1 + db1), F32)
    out = out.at[:da0, :da1].set(a)
    out = out.at[da0:, da1:].set(b)
    return out


def kernel(x, edge_index, edge_attr, params):
    n, in_ch = x.shape
    e = edge_index.shape[1]
    hid = 64
    ech = edge_attr.shape[1]

    ep = ((e + NW * CHUNK - 1) // (NW * CHUNK)) * (NW * CHUNK)
    n_rows = ((n + 1 + NS * 8 - 1) // (NS * 8)) * (NS * 8)

    src_p = jnp.zeros((ep,), jnp.int32).at[:e].set(edge_index[0])
    dst_p = jnp.full((ep,), n, jnp.int32).at[:e].set(edge_index[1])
    ea128 = jnp.zeros((ep, 128), F32)
    ea128 = ea128.at[:e, :ech].set(edge_attr)
    ea128 = ea128.at[:e, 16].set(1.0)

    c0, c1 = params["convs"][0], params["convs"][1]
    wl1 = _block_diag(c0[0]["Wl"][:64], c1[0]["Wl"][:64])
    we1 = jnp.concatenate([c0[0]["Wl"][64:], c1[0]["Wl"][64:]], axis=1)
    wr1 = _block_diag(c0[0]["Wr"], c1[0]["Wr"])
    b1v = jnp.concatenate([c0[0]["bl"] + c0[0]["br"],
                           c1[0]["bl"] + c1[0]["br"]])
    wl2 = _block_diag(c0[1]["Wl"][:64], c1[1]["Wl"][:64])
    we2 = jnp.concatenate([c0[1]["Wl"][64:], c1[1]["Wl"][64:]], axis=1)
    wr2 = _block_diag(c0[1]["Wr"], c1[1]["Wr"])
    b2v = jnp.concatenate([c0[1]["bl"] + c0[1]["br"],
                           c1[1]["bl"] + c1[1]["br"]])
    attn = params["attn"]

    eagg = _sc_edge_segsum(ea128, dst_p, n_rows)
    sa = _sc_segsum(x, src_p, dst_p, n_rows)

    h1 = _tc_layer1(sa, eagg, x, wl1, we1, wr1, b1v, n)

    sb = _sc_segsum(h1, src_p, dst_p, n_rows)

    return _tc_layer2_attn(sb, eagg, h1, wl2, we2, wr2, b2v,
                           attn["W1"], attn["b1"], attn["W2"], n)
